# baseline (speedup 1.0000x reference)
import jax
import jax.numpy as jnp
from jax import lax
from jax.experimental import pallas as pl
from jax.experimental.pallas import tpu as pltpu

E = 8
N_IN = 1024
N_HID = 2048
N_OUT = 1024
T = 2048
BT = 256


def _router_kernel(x_ref, rW1_ref, rb1_ref, rW2_ref, rb2_ref, w_ref):
    rh = jnp.dot(x_ref[...], rW1_ref[...], preferred_element_type=jnp.float32)
    rh = jnp.maximum(rh + rb1_ref[...], 0.0)
    logits = jnp.dot(rh, rW2_ref[...], preferred_element_type=jnp.float32)
    logits = logits + rb2_ref[...]
    m = jnp.max(logits, axis=-1, keepdims=True)
    ex = jnp.exp(logits - m)
    w_ref[...] = ex / jnp.sum(ex, axis=-1, keepdims=True)


def _expert_kernel(x_ref, eW1_ref, eb1_ref, eW2_ref, eb2_ref, w_ref,
                   eo_ref, comb_ref):
    e = pl.program_id(0)
    t = pl.program_id(1)
    xs = x_ref[pl.ds(t * BT, BT), :]
    eh = jnp.dot(xs, eW1_ref[0], preferred_element_type=jnp.float32)
    eh = jnp.maximum(eh + eb1_ref[0], 0.0)
    out = jnp.dot(eh, eW2_ref[0], preferred_element_type=jnp.float32)
    out = out + eb2_ref[0]
    eo_ref[...] = out

    ws = w_ref[pl.ds(t * BT, BT), :]
    sel = lax.broadcasted_iota(jnp.int32, (1, E), 1) == e
    col = jnp.sum(jnp.where(sel, ws, 0.0), axis=1, keepdims=True)
    weighted = out * col

    @pl.when(e == 0)
    def _init():
        comb_ref[pl.ds(t * BT, BT), :] = weighted

    @pl.when(e != 0)
    def _acc():
        comb_ref[pl.ds(t * BT, BT), :] += weighted


def kernel(x, rW1, rb1, rW2, rb2, eW1, eb1, eW2, eb2):
    nT = T // BT
    routing_weights = pl.pallas_call(
        _router_kernel,
        grid=(nT,),
        in_specs=[
            pl.BlockSpec((BT, N_IN), lambda t: (t, 0)),
            pl.BlockSpec((N_IN, N_HID), lambda t: (0, 0)),
            pl.BlockSpec((1, N_HID), lambda t: (0, 0)),
            pl.BlockSpec((N_HID, E), lambda t: (0, 0)),
            pl.BlockSpec((1, E), lambda t: (0, 0)),
        ],
        out_specs=pl.BlockSpec((BT, E), lambda t: (t, 0)),
        out_shape=jax.ShapeDtypeStruct((T, E), jnp.float32),
        compiler_params=pltpu.CompilerParams(
            dimension_semantics=("arbitrary",)),
    )(x, rW1, rb1.reshape(1, N_HID), rW2, rb2.reshape(1, E))

    eo_flat, combined = pl.pallas_call(
        _expert_kernel,
        grid=(E, nT),
        in_specs=[
            pl.BlockSpec((T, N_IN), lambda e, t: (0, 0)),
            pl.BlockSpec((1, N_IN, N_HID), lambda e, t: (e, 0, 0)),
            pl.BlockSpec((1, 1, N_HID), lambda e, t: (e, 0, 0)),
            pl.BlockSpec((1, N_HID, N_OUT), lambda e, t: (e, 0, 0)),
            pl.BlockSpec((1, 1, N_OUT), lambda e, t: (e, 0, 0)),
            pl.BlockSpec((T, E), lambda e, t: (0, 0)),
        ],
        out_specs=[
            pl.BlockSpec((BT, N_OUT), lambda e, t: (t, e)),
            pl.BlockSpec((T, N_OUT), lambda e, t: (0, 0)),
        ],
        out_shape=[
            jax.ShapeDtypeStruct((T, E * N_OUT), jnp.float32),
            jax.ShapeDtypeStruct((T, N_OUT), jnp.float32),
        ],
        compiler_params=pltpu.CompilerParams(
            dimension_semantics=("arbitrary", "arbitrary")),
    )(x, eW1, eb1.reshape(E, 1, N_HID), eW2, eb2.reshape(E, 1, N_OUT),
      routing_weights)

    expert_outputs = eo_flat.reshape(T, E, N_OUT)
    return (combined, routing_weights, expert_outputs)


# speedup vs baseline: 1.0068x; 1.0068x over previous
import jax
import jax.numpy as jnp
from jax import lax
from jax.experimental import pallas as pl
from jax.experimental.pallas import tpu as pltpu

E = 8
N_IN = 1024
N_HID = 2048
N_OUT = 1024
T = 2048
BT = 256


def _dot(a, b):
    return jnp.dot(a.astype(jnp.bfloat16), b.astype(jnp.bfloat16),
                   preferred_element_type=jnp.float32)


def _router_kernel(x_ref, rW1_ref, rb1_ref, rW2_ref, rb2_ref, w_ref):
    rh = jnp.maximum(_dot(x_ref[...], rW1_ref[...]) + rb1_ref[...], 0.0)
    logits = _dot(rh, rW2_ref[...])
    logits = logits + rb2_ref[...]
    m = jnp.max(logits, axis=-1, keepdims=True)
    ex = jnp.exp(logits - m)
    w_ref[...] = ex / jnp.sum(ex, axis=-1, keepdims=True)


def _expert_kernel(x_ref, eW1_ref, eb1_ref, eW2_ref, eb2_ref, w_ref,
                   eo_ref, comb_ref):
    e = pl.program_id(0)
    t = pl.program_id(1)
    xs = x_ref[pl.ds(t * BT, BT), :]
    eh = jnp.maximum(_dot(xs, eW1_ref[0]) + eb1_ref[0], 0.0)
    out = _dot(eh, eW2_ref[0]) + eb2_ref[0]
    eo_ref[...] = out

    ws = w_ref[pl.ds(t * BT, BT), :]
    sel = lax.broadcasted_iota(jnp.int32, (1, E), 1) == e
    col = jnp.sum(jnp.where(sel, ws, 0.0), axis=1, keepdims=True)
    weighted = out * col

    @pl.when(e == 0)
    def _init():
        comb_ref[pl.ds(t * BT, BT), :] = weighted

    @pl.when(e != 0)
    def _acc():
        comb_ref[pl.ds(t * BT, BT), :] += weighted


def kernel(x, rW1, rb1, rW2, rb2, eW1, eb1, eW2, eb2):
    nT = T // BT
    routing_weights = pl.pallas_call(
        _router_kernel,
        grid=(nT,),
        in_specs=[
            pl.BlockSpec((BT, N_IN), lambda t: (t, 0)),
            pl.BlockSpec((N_IN, N_HID), lambda t: (0, 0)),
            pl.BlockSpec((1, N_HID), lambda t: (0, 0)),
            pl.BlockSpec((N_HID, E), lambda t: (0, 0)),
            pl.BlockSpec((1, E), lambda t: (0, 0)),
        ],
        out_specs=pl.BlockSpec((BT, E), lambda t: (t, 0)),
        out_shape=jax.ShapeDtypeStruct((T, E), jnp.float32),
        compiler_params=pltpu.CompilerParams(
            dimension_semantics=("arbitrary",)),
    )(x, rW1, rb1.reshape(1, N_HID), rW2, rb2.reshape(1, E))

    eo_flat, combined = pl.pallas_call(
        _expert_kernel,
        grid=(E, nT),
        in_specs=[
            pl.BlockSpec((T, N_IN), lambda e, t: (0, 0)),
            pl.BlockSpec((1, N_IN, N_HID), lambda e, t: (e, 0, 0)),
            pl.BlockSpec((1, 1, N_HID), lambda e, t: (e, 0, 0)),
            pl.BlockSpec((1, N_HID, N_OUT), lambda e, t: (e, 0, 0)),
            pl.BlockSpec((1, 1, N_OUT), lambda e, t: (e, 0, 0)),
            pl.BlockSpec((T, E), lambda e, t: (0, 0)),
        ],
        out_specs=[
            pl.BlockSpec((BT, N_OUT), lambda e, t: (t, e)),
            pl.BlockSpec((T, N_OUT), lambda e, t: (0, 0)),
        ],
        out_shape=[
            jax.ShapeDtypeStruct((T, E * N_OUT), jnp.float32),
            jax.ShapeDtypeStruct((T, N_OUT), jnp.float32),
        ],
        compiler_params=pltpu.CompilerParams(
            dimension_semantics=("arbitrary", "arbitrary")),
    )(x, eW1, eb1.reshape(E, 1, N_HID), eW2, eb2.reshape(E, 1, N_OUT),
      routing_weights)

    expert_outputs = eo_flat.reshape(T, E, N_OUT)
    return (combined, routing_weights, expert_outputs)


# speedup vs baseline: 1.2378x; 1.2294x over previous
import jax
import jax.numpy as jnp
from jax import lax
from jax.experimental import pallas as pl
from jax.experimental.pallas import tpu as pltpu

E = 8
N_IN = 1024
N_HID = 2048
N_OUT = 1024
T = 2048
BT = 256


def _dot(a, b):
    return jnp.dot(a.astype(jnp.bfloat16), b.astype(jnp.bfloat16),
                   preferred_element_type=jnp.float32)


def _router_kernel(x_ref, rW1_ref, rb1_ref, rW2_ref, rb2_ref, w_ref):
    rh = jnp.maximum(_dot(x_ref[...], rW1_ref[...]) + rb1_ref[...], 0.0)
    logits = _dot(rh, rW2_ref[...])
    logits = logits + rb2_ref[...]
    m = jnp.max(logits, axis=-1, keepdims=True)
    ex = jnp.exp(logits - m)
    w_ref[...] = ex / jnp.sum(ex, axis=-1, keepdims=True)


def _expert_kernel(x_ref, eW1_ref, eb1_ref, eW2_ref, eb2_ref, w_ref,
                   eo_hbm, comb_ref, obuf, sem):
    nT = T // BT
    e = pl.program_id(0)
    t = pl.program_id(1)
    i = e * nT + t
    slot = lax.rem(i, 2)

    xs = x_ref[pl.ds(t * BT, BT), :]
    eh = jnp.maximum(_dot(xs, eW1_ref[0]) + eb1_ref[0], 0.0)
    out = _dot(eh, eW2_ref[0]) + eb2_ref[0]

    @pl.when(i >= 2)
    def _wait_old():
        i_o = i - 2
        pltpu.make_async_copy(
            obuf.at[slot],
            eo_hbm.at[pl.ds(lax.rem(i_o, nT) * BT, BT), i_o // nT],
            sem.at[slot]).wait()

    obuf[slot] = out
    pltpu.make_async_copy(
        obuf.at[slot],
        eo_hbm.at[pl.ds(t * BT, BT), e],
        sem.at[slot]).start()

    ws = w_ref[pl.ds(t * BT, BT), :]
    sel = lax.broadcasted_iota(jnp.int32, (1, E), 1) == e
    col = jnp.sum(jnp.where(sel, ws, 0.0), axis=1, keepdims=True)
    weighted = out * col

    @pl.when(e == 0)
    def _init():
        comb_ref[pl.ds(t * BT, BT), :] = weighted

    @pl.when(e != 0)
    def _acc():
        comb_ref[pl.ds(t * BT, BT), :] += weighted

    @pl.when(i == E * nT - 1)
    def _drain():
        pltpu.make_async_copy(
            obuf.at[slot],
            eo_hbm.at[pl.ds(t * BT, BT), e],
            sem.at[slot]).wait()
        i_p = i - 1
        pltpu.make_async_copy(
            obuf.at[1 - slot],
            eo_hbm.at[pl.ds(lax.rem(i_p, nT) * BT, BT), i_p // nT],
            sem.at[1 - slot]).wait()


def kernel(x, rW1, rb1, rW2, rb2, eW1, eb1, eW2, eb2):
    nT = T // BT
    routing_weights = pl.pallas_call(
        _router_kernel,
        grid=(nT,),
        in_specs=[
            pl.BlockSpec((BT, N_IN), lambda t: (t, 0)),
            pl.BlockSpec((N_IN, N_HID), lambda t: (0, 0)),
            pl.BlockSpec((1, N_HID), lambda t: (0, 0)),
            pl.BlockSpec((N_HID, E), lambda t: (0, 0)),
            pl.BlockSpec((1, E), lambda t: (0, 0)),
        ],
        out_specs=pl.BlockSpec((BT, E), lambda t: (t, 0)),
        out_shape=jax.ShapeDtypeStruct((T, E), jnp.float32),
        compiler_params=pltpu.CompilerParams(
            dimension_semantics=("arbitrary",)),
    )(x, rW1, rb1.reshape(1, N_HID), rW2, rb2.reshape(1, E))

    expert_outputs, combined = pl.pallas_call(
        _expert_kernel,
        grid=(E, nT),
        in_specs=[
            pl.BlockSpec((T, N_IN), lambda e, t: (0, 0)),
            pl.BlockSpec((1, N_IN, N_HID), lambda e, t: (e, 0, 0)),
            pl.BlockSpec((1, 1, N_HID), lambda e, t: (e, 0, 0)),
            pl.BlockSpec((1, N_HID, N_OUT), lambda e, t: (e, 0, 0)),
            pl.BlockSpec((1, 1, N_OUT), lambda e, t: (e, 0, 0)),
            pl.BlockSpec((T, E), lambda e, t: (0, 0)),
        ],
        out_specs=[
            pl.BlockSpec(memory_space=pl.ANY),
            pl.BlockSpec((T, N_OUT), lambda e, t: (0, 0)),
        ],
        out_shape=[
            jax.ShapeDtypeStruct((T, E, N_OUT), jnp.float32),
            jax.ShapeDtypeStruct((T, N_OUT), jnp.float32),
        ],
        scratch_shapes=[
            pltpu.VMEM((2, BT, N_OUT), jnp.float32),
            pltpu.SemaphoreType.DMA((2,)),
        ],
        compiler_params=pltpu.CompilerParams(
            dimension_semantics=("arbitrary", "arbitrary")),
    )(x, eW1, eb1.reshape(E, 1, N_HID), eW2, eb2.reshape(E, 1, N_OUT),
      routing_weights)

    return (combined, routing_weights, expert_outputs)
